# baseline (device time: 185861 ns/iter reference)
import functools

import jax
import jax.numpy as jnp
from jax import lax
from jax.experimental import pallas as pl
from jax.experimental.pallas import tpu as pltpu

N_DEV = 32
DH = 64


def kernel(x, Wq, K_ext, V_ext, Wo):
    B, Sq, E = x.shape
    H_local = Wq.shape[1] // DH
    Skv = K_ext.shape[1]
    rows = B * Sq
    rows_per = rows // N_DEV
    n_hops = N_DEV - 1

    my_pos = lax.axis_index("i")

    K_l = lax.dynamic_slice_in_dim(K_ext, my_pos * H_local, H_local, axis=2)
    V_l = lax.dynamic_slice_in_dim(V_ext, my_pos * H_local, H_local, axis=2)

    def body(x_ref, wq_ref, k_ref, v_ref, wo_ref, out_ref,
             comm_ref, send_sems, recv_sems):
        p = lax.axis_index("i")
        left = lax.rem(p - 1 + N_DEV, N_DEV)
        right = lax.rem(p + 1, N_DEV)

        xf = x_ref[...].reshape(rows, E).astype(jnp.bfloat16)
        wq = wq_ref[...].astype(jnp.bfloat16)
        q_all = jnp.dot(xf, wq, preferred_element_type=jnp.float32)

        qb = lax.broadcasted_iota(jnp.int32, (Sq, Skv), 0) // 64
        kb = lax.broadcasted_iota(jnp.int32, (Sq, Skv), 1) // 64
        mask = (qb == kb) | ((kb % 4) == (qb % 4))

        ctx_rows = []
        for b in range(B):
            ctx_heads = []
            for h in range(H_local):
                q = q_all[b * Sq:(b + 1) * Sq, h * DH:(h + 1) * DH]
                k = k_ref[b, :, h, :].astype(jnp.bfloat16)
                s = jnp.dot(q.astype(jnp.bfloat16), k.T,
                            preferred_element_type=jnp.float32) * 0.125
                s = jnp.where(mask, s, -1e9)
                s = s - jnp.max(s, axis=1, keepdims=True)
                e = jnp.exp(s)
                w = e / jnp.sum(e, axis=1, keepdims=True)
                v = v_ref[b, :, h, :].astype(jnp.bfloat16)
                ctx_heads.append(jnp.dot(w.astype(jnp.bfloat16), v,
                                         preferred_element_type=jnp.float32))
            ctx_rows.append(jnp.concatenate(ctx_heads, axis=1))
        ctx = jnp.concatenate(ctx_rows, axis=0)

        wo = wo_ref[...].astype(jnp.bfloat16)
        partial = jnp.dot(ctx.astype(jnp.bfloat16), wo,
                          preferred_element_type=jnp.float32)
        out_ref[...] = partial

        barrier_sem = pltpu.get_barrier_semaphore()
        for nbr in (left, right):
            pl.semaphore_signal(barrier_sem, inc=1, device_id=(nbr,),
                                device_id_type=pl.DeviceIdType.MESH)
        pl.semaphore_wait(barrier_sem, 2)

        for h in range(n_hops):
            s_chunk = lax.rem(p - h + N_DEV, N_DEV)
            r_chunk = lax.rem(p - 1 - h + 2 * N_DEV, N_DEV)
            rdma = pltpu.make_async_remote_copy(
                src_ref=out_ref.at[pl.ds(s_chunk * rows_per, rows_per)],
                dst_ref=comm_ref.at[h],
                send_sem=send_sems.at[h],
                recv_sem=recv_sems.at[h],
                device_id=(right,),
                device_id_type=pl.DeviceIdType.MESH,
            )
            rdma.start()
            rdma.wait()
            rr = pl.ds(r_chunk * rows_per, rows_per)
            out_ref[rr, :] = out_ref[rr, :] + comm_ref[h]

        for h in range(n_hops):
            g_chunk = lax.rem(p + 1 - h + 2 * N_DEV, N_DEV)
            r_chunk = lax.rem(p - h + 2 * N_DEV, N_DEV)
            slot = n_hops + h
            rdma = pltpu.make_async_remote_copy(
                src_ref=out_ref.at[pl.ds(g_chunk * rows_per, rows_per)],
                dst_ref=comm_ref.at[slot],
                send_sem=send_sems.at[slot],
                recv_sem=recv_sems.at[slot],
                device_id=(right,),
                device_id_type=pl.DeviceIdType.MESH,
            )
            rdma.start()
            rdma.wait()
            out_ref[pl.ds(r_chunk * rows_per, rows_per), :] = comm_ref[slot]

        @functools.partial(pl.run_scoped,
                           second_barrier=pltpu.SemaphoreType.REGULAR)
        def _(second_barrier):
            for nbr in (left, right):
                pl.semaphore_signal(second_barrier, inc=1, device_id=(nbr,),
                                    device_id_type=pl.DeviceIdType.MESH)
            pl.semaphore_wait(second_barrier, 2)

    n_slots = 2 * n_hops
    out = pl.pallas_call(
        body,
        out_shape=jax.ShapeDtypeStruct((rows, E), jnp.float32),
        in_specs=[pl.BlockSpec(memory_space=pltpu.VMEM)] * 5,
        out_specs=pl.BlockSpec(memory_space=pltpu.VMEM),
        scratch_shapes=[
            pltpu.VMEM((n_slots, rows_per, E), jnp.float32),
            pltpu.SemaphoreType.DMA((n_slots,)),
            pltpu.SemaphoreType.DMA((n_slots,)),
        ],
        compiler_params=pltpu.CompilerParams(collective_id=0),
    )(x, Wq, K_l, V_l, Wo)
    return out.reshape(B, Sq, E)


# device time: 91273 ns/iter; 2.0363x vs baseline; 2.0363x over previous
import functools

import jax
import jax.numpy as jnp
from jax import lax
from jax.experimental import pallas as pl
from jax.experimental.pallas import tpu as pltpu

N_DEV = 32
N_STEPS = 5
DH = 64


def kernel(x, Wq, K_ext, V_ext, Wo):
    B, Sq, E = x.shape
    H_local = Wq.shape[1] // DH
    Skv = K_ext.shape[1]
    rows = B * Sq

    my_pos = lax.axis_index("i")

    K_l = lax.dynamic_slice_in_dim(K_ext, my_pos * H_local, H_local, axis=2)
    V_l = lax.dynamic_slice_in_dim(V_ext, my_pos * H_local, H_local, axis=2)

    halves = [rows >> (k + 1) for k in range(N_STEPS)]
    slot_off = [sum(halves[:k]) for k in range(N_STEPS)]
    comm_rows = sum(halves)

    def body(x_ref, wq_ref, k_ref, v_ref, wo_ref, out_ref,
             comm_ref, send_sems, recv_sems):
        p = lax.axis_index("i")
        z = p // 8
        j = lax.rem(p, 8)
        y = j // 2
        xx = lax.rem(j + y, 2)

        def logical(x_, y_, z_):
            return 8 * z_ + 2 * y_ + lax.rem(x_ + y_, 2)

        bits = [xx, lax.rem(y, 2), y // 2, lax.rem(z, 2), z // 2]
        partners = [
            logical(1 - xx, y, z),
            logical(xx, y + 1 - 2 * bits[1], z),
            logical(xx, y + 2 - 4 * bits[2], z),
            logical(xx, y, z + 1 - 2 * bits[3]),
            logical(xx, y, z + 2 - 4 * bits[4]),
        ]

        xf = x_ref[...].reshape(rows, E).astype(jnp.bfloat16)
        wq = wq_ref[...].astype(jnp.bfloat16)
        q_all = jnp.dot(xf, wq, preferred_element_type=jnp.float32)

        qb = lax.broadcasted_iota(jnp.int32, (Sq, Skv), 0) // 64
        kb = lax.broadcasted_iota(jnp.int32, (Sq, Skv), 1) // 64
        mask = (qb == kb) | ((kb % 4) == (qb % 4))

        ctx_rows = []
        for b in range(B):
            ctx_heads = []
            for h in range(H_local):
                q = q_all[b * Sq:(b + 1) * Sq, h * DH:(h + 1) * DH]
                k = k_ref[b, :, h, :].astype(jnp.bfloat16)
                s = jnp.dot(q.astype(jnp.bfloat16), k.T,
                            preferred_element_type=jnp.float32) * 0.125
                s = jnp.where(mask, s, -1e9)
                s = s - jnp.max(s, axis=1, keepdims=True)
                e = jnp.exp(s)
                w = e / jnp.sum(e, axis=1, keepdims=True)
                v = v_ref[b, :, h, :].astype(jnp.bfloat16)
                ctx_heads.append(jnp.dot(w.astype(jnp.bfloat16), v,
                                         preferred_element_type=jnp.float32))
            ctx_rows.append(jnp.concatenate(ctx_heads, axis=1))
        ctx = jnp.concatenate(ctx_rows, axis=0)

        wo = wo_ref[...].astype(jnp.bfloat16)
        out_ref[...] = jnp.dot(ctx.astype(jnp.bfloat16), wo,
                               preferred_element_type=jnp.float32)

        barrier_sem = pltpu.get_barrier_semaphore()
        for k in range(N_STEPS):
            pl.semaphore_signal(barrier_sem, inc=1, device_id=(partners[k],),
                                device_id_type=pl.DeviceIdType.MESH)
        pl.semaphore_wait(barrier_sem, N_STEPS)

        base = jnp.int32(0)
        for k in range(N_STEPS):
            half = halves[k]
            send_start = base + (1 - bits[k]) * half
            keep_start = base + bits[k] * half
            rdma = pltpu.make_async_remote_copy(
                src_ref=out_ref.at[pl.ds(send_start, half)],
                dst_ref=comm_ref.at[pl.ds(slot_off[k], half)],
                send_sem=send_sems.at[k],
                recv_sem=recv_sems.at[k],
                device_id=(partners[k],),
                device_id_type=pl.DeviceIdType.MESH,
            )
            rdma.start()
            rdma.wait()
            kr = pl.ds(keep_start, half)
            out_ref[kr, :] = (out_ref[kr, :]
                              + comm_ref[pl.ds(slot_off[k], half), :])
            base = keep_start

        for k in reversed(range(N_STEPS)):
            size = halves[k]
            rdma = pltpu.make_async_remote_copy(
                src_ref=out_ref.at[pl.ds(base, size)],
                dst_ref=out_ref.at[pl.ds(base, size)],
                send_sem=send_sems.at[N_STEPS + k],
                recv_sem=recv_sems.at[N_STEPS + k],
                device_id=(partners[k],),
                device_id_type=pl.DeviceIdType.MESH,
            )
            rdma.start()
            rdma.wait()
            base = base - bits[k] * size

        @functools.partial(pl.run_scoped,
                           second_barrier=pltpu.SemaphoreType.REGULAR)
        def _(second_barrier):
            for k in range(N_STEPS):
                pl.semaphore_signal(second_barrier, inc=1,
                                    device_id=(partners[k],),
                                    device_id_type=pl.DeviceIdType.MESH)
            pl.semaphore_wait(second_barrier, N_STEPS)

    out = pl.pallas_call(
        body,
        out_shape=jax.ShapeDtypeStruct((rows, E), jnp.float32),
        in_specs=[pl.BlockSpec(memory_space=pltpu.VMEM)] * 5,
        out_specs=pl.BlockSpec(memory_space=pltpu.VMEM),
        scratch_shapes=[
            pltpu.VMEM((comm_rows, E), jnp.float32),
            pltpu.SemaphoreType.DMA((2 * N_STEPS,)),
            pltpu.SemaphoreType.DMA((2 * N_STEPS,)),
        ],
        compiler_params=pltpu.CompilerParams(collective_id=0),
    )(x, Wq, K_l, V_l, Wo)
    return out.reshape(B, Sq, E)
